# baseline (speedup 1.0000x reference)
import functools

import jax
import jax.numpy as jnp
from jax import lax
from jax.experimental import pallas as pl
from jax.experimental.pallas import tpu as pltpu

LATENT = 32
HIDDEN = 64


def _silu(x):
    return x * jax.nn.sigmoid(x)


def _mlp3_body(a_ref, b_ref, c_ref, w1a_ref, w1b_ref, w1c_ref, b1_ref,
               w2_ref, b2_ref, res_ref, o_ref, *, out_act):
    h = (jnp.dot(a_ref[...], w1a_ref[...], preferred_element_type=jnp.float32)
         + jnp.dot(b_ref[...], w1b_ref[...], preferred_element_type=jnp.float32)
         + jnp.dot(c_ref[...], w1c_ref[...], preferred_element_type=jnp.float32)
         + b1_ref[...])
    h = _silu(h)
    o = jnp.dot(h, w2_ref[...], preferred_element_type=jnp.float32) + b2_ref[...]
    if out_act == "silu":
        o = _silu(o)
    else:
        o = jnp.tanh(o)
    o_ref[...] = o + res_ref[...]


def _mlp3(a, b, c, W1, b1, W2, b2, res, out_act, block_n):
    n = a.shape[0]
    grid = (n + block_n - 1) // block_n
    w1a, w1b, w1c = W1[:LATENT], W1[LATENT:2 * LATENT], W1[2 * LATENT:]
    row_spec = pl.BlockSpec((block_n, LATENT), lambda i: (i, 0))
    full = lambda s: pl.BlockSpec(s, lambda i: (0,) * len(s))
    return pl.pallas_call(
        functools.partial(_mlp3_body, out_act=out_act),
        grid=(grid,),
        in_specs=[row_spec, row_spec, row_spec,
                  full((LATENT, HIDDEN)), full((LATENT, HIDDEN)),
                  full((LATENT, HIDDEN)), full((1, HIDDEN)),
                  full((HIDDEN, LATENT)), full((1, LATENT)), row_spec],
        out_specs=row_spec,
        out_shape=jax.ShapeDtypeStruct((n, LATENT), jnp.float32),
    )(a, b, c, w1a, w1b, w1c, b1.reshape(1, HIDDEN), W2,
      b2.reshape(1, LATENT), res)


def _score_body(x_ref, w1_ref, b1_ref, w2_ref, b2_ref, lg_ref, o_ref):
    h = jnp.tanh(jnp.dot(x_ref[...], w1_ref[...],
                         preferred_element_type=jnp.float32) + b1_ref[...])
    s = jnp.dot(h, w2_ref[...], preferred_element_type=jnp.float32) + b2_ref[...]
    o_ref[...] = jnp.exp(lg_ref[...] + s)


def _bs_scores(x, W1, b1, W2, b2, logits, block_n):
    n = x.shape[0]
    grid = (n + block_n - 1) // block_n
    full = lambda s: pl.BlockSpec(s, lambda i: (0,) * len(s))
    return pl.pallas_call(
        _score_body,
        grid=(grid,),
        in_specs=[pl.BlockSpec((block_n, 2 * LATENT), lambda i: (i, 0)),
                  full((2 * LATENT, HIDDEN)), full((1, HIDDEN)),
                  full((HIDDEN, 1)), full((1, 1)),
                  pl.BlockSpec((block_n, 1), lambda i: (i, 0))],
        out_specs=pl.BlockSpec((block_n, 1), lambda i: (i, 0)),
        out_shape=jax.ShapeDtypeStruct((n, 1), jnp.float32),
    )(x, W1, b1.reshape(1, HIDDEN), W2, b2.reshape(1, 1),
      logits.reshape(n, 1))


def kernel(nodes, edges, supernodes, superedges, graph, bipartite_graph,
           bipartite_graph_attention_logits, super_graph,
           super_graph_attention, en_W1, en_b1, en_W2, en_b2, nn_W1, nn_b1,
           nn_W2, nn_b2, sn_W1, sn_b1, sn_W2, sn_b2, se_W1, se_b1, se_W2,
           se_b2, bs_W1, bs_b1, bs_W2, bs_b2):
    g0, g1 = graph[0], graph[1]
    bg0, bg1 = bipartite_graph[0], bipartite_graph[1]
    sg0, sg1 = super_graph[0], super_graph[1]
    n_nodes = nodes.shape[0]
    n_super = supernodes.shape[0]

    bx = jnp.concatenate([nodes[bg0], supernodes[bg1]], axis=-1)
    att = _bs_scores(bx, bs_W1, bs_b1, bs_W2, bs_b2,
                     bipartite_graph_attention_logits, 4000)[:, 0]
    denom = jax.ops.segment_sum(att, bg0, num_segments=n_nodes)
    att = att / (1e-12 + denom[bg0])
    att = att[:, None]

    node_messages = jax.ops.segment_sum(att * nodes[bg0], bg1,
                                        num_segments=n_super)
    attention_messages = jax.ops.segment_sum(
        superedges[sg0] * super_graph_attention, sg1, num_segments=n_super)
    supernodes = _mlp3(supernodes, attention_messages, node_messages,
                       sn_W1, sn_b1, sn_W2, sn_b2, supernodes, "silu", 1000)

    supernode_messages = jax.ops.segment_sum(att * supernodes[bg1], bg0,
                                             num_segments=n_nodes)
    edge_messages = jax.ops.segment_sum(edges, g1, num_segments=n_nodes)
    nodes = _mlp3(nodes, edge_messages, supernode_messages,
                  nn_W1, nn_b1, nn_W2, nn_b2, nodes, "silu", 5000)

    superedges = _mlp3(supernodes[sg0], supernodes[sg1], superedges,
                       se_W1, se_b1, se_W2, se_b2, superedges, "tanh", 2000)

    edges = _mlp3(nodes[g0], nodes[g1], edges,
                  en_W1, en_b1, en_W2, en_b2, edges, "tanh", 4000)

    return (nodes, edges, supernodes, superedges)


# speedup vs baseline: 1.3962x; 1.3962x over previous
import functools

import jax
import jax.numpy as jnp
from jax import lax
from jax.experimental import pallas as pl
from jax.experimental.pallas import tpu as pltpu
from jax.experimental.pallas import tpu_sc as plsc

LATENT = 32
HIDDEN = 64
NC, NS = 2, 16
NW = NC * NS


def _pad_rows(x, mult, fill=0):
    n = x.shape[0]
    pad = (-n) % mult
    if pad == 0:
        return x
    return jnp.concatenate(
        [x, jnp.full((pad,) + x.shape[1:], fill, x.dtype)], axis=0)


@functools.partial(jax.jit, static_argnames=("chunk",))
def _sc_gather(table, idx, *, chunk):
    B = idx.shape[0]
    D = table.shape[1]
    b_per_w = B // NW
    iters = b_per_w // chunk
    mesh = plsc.VectorSubcoreMesh(core_axis_name="c", subcore_axis_name="s")

    @functools.partial(
        pl.kernel, mesh=mesh,
        out_type=jax.ShapeDtypeStruct((B, D), jnp.float32),
        scratch_types=[pltpu.VMEM((chunk,), jnp.int32),
                       pltpu.VMEM((chunk, D), jnp.float32),
                       pltpu.SemaphoreType.DMA],
        compiler_params=pltpu.CompilerParams(use_tc_tiling_on_sc=False),
    )
    def k(table_hbm, idx_hbm, out_hbm, idx_v, rows_v, sem):
        wid = lax.axis_index("s") * NC + lax.axis_index("c")

        def body(i, _):
            base = wid * b_per_w + i * chunk
            pltpu.sync_copy(idx_hbm.at[pl.ds(base, chunk)], idx_v)
            pltpu.async_copy(table_hbm.at[idx_v], rows_v, sem).wait()
            pltpu.sync_copy(rows_v, out_hbm.at[pl.ds(base, chunk)])
            return 0

        lax.fori_loop(0, iters, body, 0)

    return k(table, idx)


def _silu(x):
    return x * jax.nn.sigmoid(x)


def _mlp3_body(a_ref, b_ref, c_ref, w1a_ref, w1b_ref, w1c_ref, b1_ref,
               w2_ref, b2_ref, res_ref, o_ref, *, out_act):
    h = (jnp.dot(a_ref[...], w1a_ref[...], preferred_element_type=jnp.float32)
         + jnp.dot(b_ref[...], w1b_ref[...], preferred_element_type=jnp.float32)
         + jnp.dot(c_ref[...], w1c_ref[...], preferred_element_type=jnp.float32)
         + b1_ref[...])
    h = _silu(h)
    o = jnp.dot(h, w2_ref[...], preferred_element_type=jnp.float32) + b2_ref[...]
    if out_act == "silu":
        o = _silu(o)
    else:
        o = jnp.tanh(o)
    o_ref[...] = o + res_ref[...]


def _mlp3(a, b, c, W1, b1, W2, b2, res, out_act, block_n):
    n = a.shape[0]
    grid = (n + block_n - 1) // block_n
    w1a, w1b, w1c = W1[:LATENT], W1[LATENT:2 * LATENT], W1[2 * LATENT:]
    row_spec = pl.BlockSpec((block_n, LATENT), lambda i: (i, 0))
    full = lambda s: pl.BlockSpec(s, lambda i: (0,) * len(s))
    return pl.pallas_call(
        functools.partial(_mlp3_body, out_act=out_act),
        grid=(grid,),
        in_specs=[row_spec, row_spec, row_spec,
                  full((LATENT, HIDDEN)), full((LATENT, HIDDEN)),
                  full((LATENT, HIDDEN)), full((1, HIDDEN)),
                  full((HIDDEN, LATENT)), full((1, LATENT)), row_spec],
        out_specs=row_spec,
        out_shape=jax.ShapeDtypeStruct((n, LATENT), jnp.float32),
    )(a, b, c, w1a, w1b, w1c, b1.reshape(1, HIDDEN), W2,
      b2.reshape(1, LATENT), res)


def _score_body(x_ref, w1_ref, b1_ref, w2_ref, b2_ref, lg_ref, o_ref):
    h = jnp.tanh(jnp.dot(x_ref[...], w1_ref[...],
                         preferred_element_type=jnp.float32) + b1_ref[...])
    s = jnp.dot(h, w2_ref[...], preferred_element_type=jnp.float32) + b2_ref[...]
    o_ref[...] = jnp.exp(lg_ref[...] + s)


def _bs_scores(x, W1, b1, W2, b2, logits, block_n):
    n = x.shape[0]
    grid = (n + block_n - 1) // block_n
    full = lambda s: pl.BlockSpec(s, lambda i: (0,) * len(s))
    return pl.pallas_call(
        _score_body,
        grid=(grid,),
        in_specs=[pl.BlockSpec((block_n, 2 * LATENT), lambda i: (i, 0)),
                  full((2 * LATENT, HIDDEN)), full((1, HIDDEN)),
                  full((HIDDEN, 1)), full((1, 1)),
                  pl.BlockSpec((block_n, 1), lambda i: (i, 0))],
        out_specs=pl.BlockSpec((block_n, 1), lambda i: (i, 0)),
        out_shape=jax.ShapeDtypeStruct((n, 1), jnp.float32),
    )(x, W1, b1.reshape(1, HIDDEN), W2, b2.reshape(1, 1),
      logits.reshape(n, 1))


def kernel(nodes, edges, supernodes, superedges, graph, bipartite_graph,
           bipartite_graph_attention_logits, super_graph,
           super_graph_attention, en_W1, en_b1, en_W2, en_b2, nn_W1, nn_b1,
           nn_W2, nn_b2, sn_W1, sn_b1, sn_W2, sn_b2, se_W1, se_b1, se_W2,
           se_b2, bs_W1, bs_b1, bs_W2, bs_b2):
    g0, g1 = graph[0], graph[1]
    bg0, bg1 = bipartite_graph[0], bipartite_graph[1]
    sg0, sg1 = super_graph[0], super_graph[1]
    n_nodes = nodes.shape[0]
    n_super = supernodes.shape[0]

    bg0p = _pad_rows(bg0, NW * 1600)
    bg1p = _pad_rows(bg1, NW * 1600)
    nb0 = _sc_gather(nodes, bg0p, chunk=1600)[:bg0.shape[0]]
    sb1 = _sc_gather(supernodes, bg1p, chunk=1600)[:bg1.shape[0]]

    bx = jnp.concatenate([nb0, sb1], axis=-1)
    att = _bs_scores(bx, bs_W1, bs_b1, bs_W2, bs_b2,
                     bipartite_graph_attention_logits, 4000)[:, 0]
    denom = jax.ops.segment_sum(att, bg0, num_segments=n_nodes)
    att = att / (1e-12 + denom[bg0])
    att = att[:, None]

    node_messages = jax.ops.segment_sum(att * nb0, bg1,
                                        num_segments=n_super)
    attention_messages = jax.ops.segment_sum(
        superedges[sg0] * super_graph_attention, sg1, num_segments=n_super)
    supernodes = _mlp3(supernodes, attention_messages, node_messages,
                       sn_W1, sn_b1, sn_W2, sn_b2, supernodes, "silu", 1000)

    sup_b1 = _sc_gather(supernodes, bg1p, chunk=1600)[:bg1.shape[0]]
    supernode_messages = jax.ops.segment_sum(att * sup_b1, bg0,
                                             num_segments=n_nodes)
    edge_messages = jax.ops.segment_sum(edges, g1, num_segments=n_nodes)
    nodes = _mlp3(nodes, edge_messages, supernode_messages,
                  nn_W1, nn_b1, nn_W2, nn_b2, nodes, "silu", 5000)

    sg0p = _pad_rows(sg0, NW * 512)
    sg1p = _pad_rows(sg1, NW * 512)
    sup_s0 = _sc_gather(supernodes, sg0p, chunk=512)[:sg0.shape[0]]
    sup_s1 = _sc_gather(supernodes, sg1p, chunk=512)[:sg1.shape[0]]
    superedges = _mlp3(sup_s0, sup_s1, superedges,
                       se_W1, se_b1, se_W2, se_b2, superedges, "tanh", 2000)

    x0 = _sc_gather(nodes, g0, chunk=1000)
    x1 = _sc_gather(nodes, g1, chunk=1000)
    edges = _mlp3(x0, x1, edges,
                  en_W1, en_b1, en_W2, en_b2, edges, "tanh", 4000)

    return (nodes, edges, supernodes, superedges)


# speedup vs baseline: 2.6630x; 1.9073x over previous
import functools

import jax
import jax.numpy as jnp
from jax import lax
from jax.experimental import pallas as pl
from jax.experimental.pallas import tpu as pltpu
from jax.experimental.pallas import tpu_sc as plsc

LATENT = 32
HIDDEN = 64
NC, NS = 2, 16
NW = NC * NS


def _pad_rows(x, mult, fill=0):
    n = x.shape[0]
    pad = (-n) % mult
    if pad == 0:
        return x
    return jnp.concatenate(
        [x, jnp.full((pad,) + x.shape[1:], fill, x.dtype)], axis=0)


@functools.partial(jax.jit, static_argnames=("chunk",))
def _sc_gather(table, idx, *, chunk):
    B = idx.shape[0]
    D = table.shape[1]
    b_per_w = B // NW
    iters = b_per_w // chunk
    mesh = plsc.VectorSubcoreMesh(core_axis_name="c", subcore_axis_name="s")

    @functools.partial(
        pl.kernel, mesh=mesh,
        out_type=jax.ShapeDtypeStruct((B, D), jnp.float32),
        scratch_types=[pltpu.VMEM((chunk,), jnp.int32),
                       pltpu.VMEM((chunk, D), jnp.float32),
                       pltpu.SemaphoreType.DMA],
        compiler_params=pltpu.CompilerParams(use_tc_tiling_on_sc=False),
    )
    def k(table_hbm, idx_hbm, out_hbm, idx_v, rows_v, sem):
        wid = lax.axis_index("s") * NC + lax.axis_index("c")

        def body(i, _):
            base = wid * b_per_w + i * chunk
            pltpu.sync_copy(idx_hbm.at[pl.ds(base, chunk)], idx_v)
            pltpu.async_copy(table_hbm.at[idx_v], rows_v, sem).wait()
            pltpu.sync_copy(rows_v, out_hbm.at[pl.ds(base, chunk)])
            return 0

        lax.fori_loop(0, iters, body, 0)

    return k(table, idx)


def _silu(x):
    return x * jax.nn.sigmoid(x)


@functools.partial(jax.jit, static_argnames=("n_seg_pad", "chunk"))
def _sc_scatter_add(vals, idx, *, n_seg_pad, chunk):
    B, D = vals.shape
    b_per_w = B // NW
    iters = b_per_w // chunk
    z = n_seg_pad // NS
    mesh = plsc.VectorSubcoreMesh(core_axis_name="c", subcore_axis_name="s")
    zeros = jnp.zeros((n_seg_pad, D), jnp.float32)

    @functools.partial(
        pl.kernel, mesh=mesh,
        out_type=jax.ShapeDtypeStruct((NC, n_seg_pad, D), jnp.float32),
        scratch_types=[pltpu.VMEM((chunk,), jnp.int32),
                       pltpu.VMEM((chunk, D), jnp.float32),
                       pltpu.VMEM_SHARED((n_seg_pad, D), jnp.float32)],
        compiler_params=pltpu.CompilerParams(use_tc_tiling_on_sc=False),
    )
    def k(vals_hbm, idx_hbm, zeros_hbm, out_hbm, idx_v, rows_v, acc_sh):
        cid = lax.axis_index("c")
        sid = lax.axis_index("s")
        pltpu.sync_copy(zeros_hbm.at[pl.ds(sid * z, z)],
                        acc_sh.at[pl.ds(sid * z, z)])
        plsc.subcore_barrier()

        def body(i, _):
            base = (sid * NC + cid) * b_per_w + i * chunk
            pltpu.sync_copy(idx_hbm.at[pl.ds(base, chunk)], idx_v)
            pltpu.sync_copy(vals_hbm.at[pl.ds(base, chunk)], rows_v)
            pltpu.sync_copy(rows_v, acc_sh.at[idx_v], add=True)
            return 0

        lax.fori_loop(0, iters, body, 0)
        plsc.subcore_barrier()
        pltpu.sync_copy(acc_sh.at[pl.ds(sid * z, z)],
                        out_hbm.at[cid].at[pl.ds(sid * z, z)])

    return k(vals, idx, zeros)


def _mlp3_body(a_ref, b_ref, c_ref, w1a_ref, w1b_ref, w1c_ref, b1_ref,
               w2_ref, b2_ref, res_ref, o_ref, *, out_act):
    h = (jnp.dot(a_ref[...], w1a_ref[...], preferred_element_type=jnp.float32)
         + jnp.dot(b_ref[...], w1b_ref[...], preferred_element_type=jnp.float32)
         + jnp.dot(c_ref[...], w1c_ref[...], preferred_element_type=jnp.float32)
         + b1_ref[...])
    h = _silu(h)
    o = jnp.dot(h, w2_ref[...], preferred_element_type=jnp.float32) + b2_ref[...]
    if out_act == "silu":
        o = _silu(o)
    else:
        o = jnp.tanh(o)
    o_ref[...] = o + res_ref[...]


def _mlp3(a, b, c, W1, b1, W2, b2, res, out_act, block_n):
    n = a.shape[0]
    grid = (n + block_n - 1) // block_n
    w1a, w1b, w1c = W1[:LATENT], W1[LATENT:2 * LATENT], W1[2 * LATENT:]
    row_spec = pl.BlockSpec((block_n, LATENT), lambda i: (i, 0))
    full = lambda s: pl.BlockSpec(s, lambda i: (0,) * len(s))
    return pl.pallas_call(
        functools.partial(_mlp3_body, out_act=out_act),
        grid=(grid,),
        in_specs=[row_spec, row_spec, row_spec,
                  full((LATENT, HIDDEN)), full((LATENT, HIDDEN)),
                  full((LATENT, HIDDEN)), full((1, HIDDEN)),
                  full((HIDDEN, LATENT)), full((1, LATENT)), row_spec],
        out_specs=row_spec,
        out_shape=jax.ShapeDtypeStruct((n, LATENT), jnp.float32),
    )(a, b, c, w1a, w1b, w1c, b1.reshape(1, HIDDEN), W2,
      b2.reshape(1, LATENT), res)


def _mlp3p_body(a_ref, bp_ref, cp_ref, w1a_ref, w1b_ref, w1c_ref, b1_ref,
                w2_ref, b2_ref, res_ref, o_ref, *, out_act):
    b = bp_ref[0] + bp_ref[1]
    c = cp_ref[0] + cp_ref[1]
    h = (jnp.dot(a_ref[...], w1a_ref[...], preferred_element_type=jnp.float32)
         + jnp.dot(b, w1b_ref[...], preferred_element_type=jnp.float32)
         + jnp.dot(c, w1c_ref[...], preferred_element_type=jnp.float32)
         + b1_ref[...])
    h = _silu(h)
    o = jnp.dot(h, w2_ref[...], preferred_element_type=jnp.float32) + b2_ref[...]
    if out_act == "silu":
        o = _silu(o)
    else:
        o = jnp.tanh(o)
    o_ref[...] = o + res_ref[...]


def _mlp3p(a, bpair, cpair, W1, b1, W2, b2, res, out_act, block_n):
    n = a.shape[0]
    grid = (n + block_n - 1) // block_n
    w1a, w1b, w1c = W1[:LATENT], W1[LATENT:2 * LATENT], W1[2 * LATENT:]
    row_spec = pl.BlockSpec((block_n, LATENT), lambda i: (i, 0))
    pair_spec = pl.BlockSpec((2, block_n, LATENT), lambda i: (0, i, 0))
    full = lambda s: pl.BlockSpec(s, lambda i: (0,) * len(s))
    return pl.pallas_call(
        functools.partial(_mlp3p_body, out_act=out_act),
        grid=(grid,),
        in_specs=[row_spec, pair_spec, pair_spec,
                  full((LATENT, HIDDEN)), full((LATENT, HIDDEN)),
                  full((LATENT, HIDDEN)), full((1, HIDDEN)),
                  full((HIDDEN, LATENT)), full((1, LATENT)), row_spec],
        out_specs=row_spec,
        out_shape=jax.ShapeDtypeStruct((n, LATENT), jnp.float32),
    )(a, bpair, cpair, w1a, w1b, w1c, b1.reshape(1, HIDDEN), W2,
      b2.reshape(1, LATENT), res)


def _score_body(a_ref, b_ref, w1a_ref, w1b_ref, b1_ref, w2_ref, b2_ref,
                lg_ref, o_ref):
    h = jnp.tanh(
        jnp.dot(a_ref[...], w1a_ref[...], preferred_element_type=jnp.float32)
        + jnp.dot(b_ref[...], w1b_ref[...], preferred_element_type=jnp.float32)
        + b1_ref[...])
    s = jnp.dot(h, w2_ref[...], preferred_element_type=jnp.float32) + b2_ref[...]
    att = jnp.exp(lg_ref[...] + s)
    col = lax.broadcasted_iota(jnp.int32, o_ref.shape, 1)
    o_ref[...] = jnp.where(col == 0, att, 0.0)


def _bs_scores(a, b, W1, b1, W2, b2, logits, block_n):
    n = a.shape[0]
    grid = (n + block_n - 1) // block_n
    full = lambda s: pl.BlockSpec(s, lambda i: (0,) * len(s))
    row_spec = pl.BlockSpec((block_n, LATENT), lambda i: (i, 0))
    return pl.pallas_call(
        _score_body,
        grid=(grid,),
        in_specs=[row_spec, row_spec,
                  full((LATENT, HIDDEN)), full((LATENT, HIDDEN)),
                  full((1, HIDDEN)), full((HIDDEN, 1)), full((1, 1)),
                  pl.BlockSpec((block_n, 1), lambda i: (i, 0))],
        out_specs=row_spec,
        out_shape=jax.ShapeDtypeStruct((n, LATENT), jnp.float32),
    )(a, b, W1[:LATENT], W1[LATENT:], b1.reshape(1, HIDDEN), W2,
      b2.reshape(1, 1), logits.reshape(n, 1))


def _pair_add_body(p_ref, o_ref):
    o_ref[...] = p_ref[0] + p_ref[1]


def _pair_add(p, block_n):
    n = p.shape[1]
    grid = (n + block_n - 1) // block_n
    return pl.pallas_call(
        _pair_add_body,
        grid=(grid,),
        in_specs=[pl.BlockSpec((2, block_n, LATENT), lambda i: (0, i, 0))],
        out_specs=pl.BlockSpec((block_n, LATENT), lambda i: (i, 0)),
        out_shape=jax.ShapeDtypeStruct((n, LATENT), jnp.float32),
    )(p)


def _rowscale_body(a_ref, s_ref, o_ref):
    o_ref[...] = a_ref[...] * s_ref[...]


def _rowscale(a, s, block_n):
    n = a.shape[0]
    grid = (n + block_n - 1) // block_n
    return pl.pallas_call(
        _rowscale_body,
        grid=(grid,),
        in_specs=[pl.BlockSpec((block_n, LATENT), lambda i: (i, 0)),
                  pl.BlockSpec((block_n, 1), lambda i: (i, 0))],
        out_specs=pl.BlockSpec((block_n, LATENT), lambda i: (i, 0)),
        out_shape=jax.ShapeDtypeStruct((n, LATENT), jnp.float32),
    )(a, s)


def _attn_body(am_ref, dr_ref, nb_ref, attn_ref, vnm_ref):
    att0 = am_ref[:, 0:1]
    dn = dr_ref[:, 0:1]
    attn = att0 / (1e-12 + dn)
    attn_ref[...] = attn
    vnm_ref[...] = attn * nb_ref[...]


def _attn_combine(attmat, drows, nb0, block_n):
    n = attmat.shape[0]
    grid = (n + block_n - 1) // block_n
    row_spec = pl.BlockSpec((block_n, LATENT), lambda i: (i, 0))
    return pl.pallas_call(
        _attn_body,
        grid=(grid,),
        in_specs=[row_spec, row_spec, row_spec],
        out_specs=(pl.BlockSpec((block_n, 1), lambda i: (i, 0)), row_spec),
        out_shape=(jax.ShapeDtypeStruct((n, 1), jnp.float32),
                   jax.ShapeDtypeStruct((n, LATENT), jnp.float32)),
    )(attmat, drows, nb0)


def kernel(nodes, edges, supernodes, superedges, graph, bipartite_graph,
           bipartite_graph_attention_logits, super_graph,
           super_graph_attention, en_W1, en_b1, en_W2, en_b2, nn_W1, nn_b1,
           nn_W2, nn_b2, sn_W1, sn_b1, sn_W2, sn_b2, se_W1, se_b1, se_W2,
           se_b2, bs_W1, bs_b1, bs_W2, bs_b2):
    g0, g1 = graph[0], graph[1]
    bg0, bg1 = bipartite_graph[0], bipartite_graph[1]
    sg0, sg1 = super_graph[0], super_graph[1]
    n_nodes = nodes.shape[0]
    n_super = supernodes.shape[0]
    NPAD = 50048
    SPAD = 1024

    bg0g = _pad_rows(bg0, NW * 1600, 0)
    bg0s = _pad_rows(bg0, NW * 1600, NPAD - 1)
    bg1g = _pad_rows(bg1, NW * 1600, 0)
    bg1s = _pad_rows(bg1, NW * 1600, SPAD - 1)
    sg0g = _pad_rows(sg0, NW * 512, 0)
    sg1g = _pad_rows(sg1, NW * 512, 0)
    sg1s = _pad_rows(sg1, NW * 512, SPAD - 1)

    nb0 = _sc_gather(nodes, bg0g, chunk=1600)
    sb1 = _sc_gather(supernodes, bg1g, chunk=1600)
    lg = _pad_rows(bipartite_graph_attention_logits, NW * 1600)
    attmat = _bs_scores(nb0, sb1, bs_W1, bs_b1, bs_W2, bs_b2, lg, 6400)
    dpair = _sc_scatter_add(attmat, bg0s, n_seg_pad=NPAD, chunk=800)
    dmat = _pair_add(dpair, 6256)
    drows = _sc_gather(dmat, bg0g, chunk=1600)
    attn, vals_nm = _attn_combine(attmat, drows, nb0, 6400)

    nm_pair = _sc_scatter_add(vals_nm, bg1s, n_seg_pad=SPAD, chunk=1600)
    se0 = _sc_gather(superedges, sg0g, chunk=512)
    sga = _pad_rows(super_graph_attention, NW * 512)
    vals_am = _rowscale(se0, sga, 2048)
    am_pair = _sc_scatter_add(vals_am, sg1s, n_seg_pad=SPAD, chunk=512)
    supernodes = _mlp3p(supernodes, am_pair, nm_pair,
                        sn_W1, sn_b1, sn_W2, sn_b2, supernodes, "silu", 1000)

    sup_b1 = _sc_gather(supernodes, bg1g, chunk=1600)
    vals_sm = _rowscale(sup_b1, attn, 6400)
    sm_pair = _sc_scatter_add(vals_sm, bg0s, n_seg_pad=NPAD, chunk=800)
    em_pair = _sc_scatter_add(edges, g1, n_seg_pad=NPAD, chunk=200)
    nodes = _mlp3p(nodes, em_pair, sm_pair,
                   nn_W1, nn_b1, nn_W2, nn_b2, nodes, "silu", 5000)

    sup_s0 = _sc_gather(supernodes, sg0g, chunk=512)[:sg0.shape[0]]
    sup_s1 = _sc_gather(supernodes, sg1g, chunk=512)[:sg1.shape[0]]
    superedges = _mlp3(sup_s0, sup_s1, superedges,
                       se_W1, se_b1, se_W2, se_b2, superedges, "tanh", 2000)

    x0 = _sc_gather(nodes, g0, chunk=1000)
    x1 = _sc_gather(nodes, g1, chunk=1000)
    edges = _mlp3(x0, x1, edges,
                  en_W1, en_b1, en_W2, en_b2, edges, "tanh", 4000)

    return (nodes, edges, supernodes, superedges)
